# Initial kernel scaffold; baseline (speedup 1.0000x reference)
import jax
import jax.numpy as jnp
from jax.experimental import pallas as pl


def kernel(input, signals_weight, channels_weight, timestamps_weight, permute_hv):
    raise NotImplementedError("write your pallas kernel here")


# speedup vs baseline: 11.5594x; 11.5594x over previous
import jax
import jax.numpy as jnp
from jax.experimental import pallas as pl

B, T, C, D = 16, 128, 16, 2048
L = 21
N = 4
TP = T - (N - 1)
BT = B * T
DC = 512


def _tc_body(inp_ref, sw_ref, tw_ref, pm_ref, out_ref):
    x = inp_ref[...]
    lev = jnp.round((x - 0.0) / 20.0 * 20.0)
    idx = jnp.clip(lev, 0, L - 1).astype(jnp.int32)
    levels = jax.lax.broadcasted_iota(jnp.int32, (1, L), 1)
    counts = jnp.zeros((BT, L), jnp.float32)
    for c in range(C):
        counts = counts + (idx[:, c:c + 1] == levels).astype(jnp.float32)
    s = jnp.dot(counts, sw_ref[...], preferred_element_type=jnp.float32)
    tw = tw_ref[...]
    twf = jnp.broadcast_to(tw[:, None, :], (T, B, DC)).reshape(BT, DC)
    samples = s * twf
    g = (samples[0:TP * B]
         * samples[B:(TP + 1) * B]
         * samples[2 * B:(TP + 2) * B]
         * samples[3 * B:(TP + 3) * B])
    acc = jnp.sum(g.reshape(TP, B, DC), axis=0)
    p = pm_ref[0, :] * pm_ref[1, :] * pm_ref[2, :]
    v = acc * p[None, :]
    out_ref[...] = jnp.where(v > 0, 1.0, -1.0)


def kernel(input, signals_weight, channels_weight, timestamps_weight, permute_hv):
    del channels_weight
    inp2 = jnp.transpose(input, (1, 0, 2)).reshape(BT, C)
    return pl.pallas_call(
        _tc_body,
        grid=(D // DC,),
        in_specs=[
            pl.BlockSpec((BT, C), lambda d: (0, 0)),
            pl.BlockSpec((L, DC), lambda d: (0, d)),
            pl.BlockSpec((T, DC), lambda d: (0, d)),
            pl.BlockSpec((N - 1, DC), lambda d: (0, d)),
        ],
        out_specs=pl.BlockSpec((B, DC), lambda d: (0, d)),
        out_shape=jax.ShapeDtypeStruct((B, D), jnp.float32),
    )(inp2, signals_weight, timestamps_weight, permute_hv)
